# baseline (speedup 1.0000x reference)
import jax
import jax.numpy as jnp
from jax import lax
from jax.experimental import pallas as pl
from jax.experimental.pallas import tpu as pltpu
from jax.experimental.pallas import tpu_sc as plsc

VOCAB = 100000
EMBED = 128
SEQ = 10
HIDDEN = 1024
BATCH = 16384
NUM_IDX = BATCH * SEQ

NC, NS = 2, 16
NW = NC * NS

SLICES = (4096, 4096, 4096, 4096)
GROUND = 320


def _make_gather_body(b_per_w):
    nround = b_per_w // GROUND

    def _gather_body(table_hbm, idx_hbm, out_hbm, idx_v, rows_a, rows_b,
                     gsem_a, gsem_b, wsem_a, wsem_b):
        wid = lax.axis_index("s") * NC + lax.axis_index("c")
        base = wid * b_per_w

        pltpu.sync_copy(idx_hbm.at[pl.ds(base, b_per_w)], idx_v)

        bufs = [(rows_a, gsem_a, wsem_a), (rows_b, gsem_b, wsem_b)]
        gather = {}
        write = [None, None]

        def start_gather(k):
            rv, gs, _ = bufs[k % 2]
            gather[k] = pltpu.async_copy(
                table_hbm.at[idx_v.at[pl.ds(k * GROUND, GROUND)]], rv, gs)

        start_gather(0)
        for k in range(nround):
            rv, _, ws = bufs[k % 2]
            if k + 1 < nround:
                if write[(k + 1) % 2] is not None:
                    write[(k + 1) % 2].wait()
                start_gather(k + 1)
            gather[k].wait()
            write[k % 2] = pltpu.async_copy(
                rv, out_hbm.at[pl.ds(base + k * GROUND, GROUND)], ws)
        for w in write:
            if w is not None:
                w.wait()

    return _gather_body


def _sc_gather(table, idx):
    n_idx = idx.shape[0]
    b_per_w = n_idx // NW
    mesh = plsc.VectorSubcoreMesh(core_axis_name="c", subcore_axis_name="s")
    kfn = pl.kernel(
        _make_gather_body(b_per_w),
        mesh=mesh,
        out_type=jax.ShapeDtypeStruct((n_idx, EMBED), table.dtype),
        scratch_types=[
            pltpu.VMEM((b_per_w,), jnp.int32),
            pltpu.VMEM((GROUND, EMBED), table.dtype),
            pltpu.VMEM((GROUND, EMBED), table.dtype),
            pltpu.SemaphoreType.DMA,
            pltpu.SemaphoreType.DMA,
            pltpu.SemaphoreType.DMA,
            pltpu.SemaphoreType.DMA,
        ],
    )
    return kfn(table, idx)


def _mlp_body(g_ref, w1t_ref, b1_ref, w2c_ref, b2_ref, out_ref):
    a = jnp.concatenate([g_ref[s] for s in range(SEQ)], axis=-1)
    a = a.astype(jnp.bfloat16)
    h = jnp.dot(a, w1t_ref[...], preferred_element_type=jnp.float32)
    hb = jnp.maximum(h.astype(jnp.bfloat16) + b1_ref[...], 0)
    s128 = jnp.dot(hb, w2c_ref[...], preferred_element_type=jnp.float32)
    s = s128[:, 0] + b2_ref[0, 0]
    out_ref[...] = jax.nn.sigmoid(s)


def _tc_mlp(g3, w1t, b1, w2c, b2):
    batch_s = g3.shape[1]
    chunk_m = min(2048, batch_s)
    return pl.pallas_call(
        _mlp_body,
        grid=(batch_s // chunk_m,),
        in_specs=[
            pl.BlockSpec((SEQ, chunk_m, EMBED), lambda i: (0, i, 0)),
            pl.BlockSpec((SEQ * EMBED, HIDDEN), lambda i: (0, 0)),
            pl.BlockSpec((1, HIDDEN), lambda i: (0, 0)),
            pl.BlockSpec((HIDDEN, 128), lambda i: (0, 0)),
            pl.BlockSpec((1, 1), lambda i: (0, 0)),
        ],
        out_specs=pl.BlockSpec((chunk_m,), lambda i: (i,)),
        out_shape=jax.ShapeDtypeStruct((batch_s,), jnp.float32),
        compiler_params=pltpu.CompilerParams(
            dimension_semantics=("parallel",),
        ),
    )(g3, w1t, b1, w2c, b2)


def kernel(x, table, W1, b1, W2, b2):
    w1t = W1.T.astype(jnp.bfloat16)
    b1r = b1.astype(jnp.bfloat16).reshape(1, HIDDEN)
    w2c = jnp.zeros((HIDDEN, 128), jnp.float32).at[:, 0].set(W2[0])
    w2c = w2c.astype(jnp.bfloat16)
    b2r = b2.reshape(1, 1)
    outs = []
    start = 0
    for bs in SLICES:
        xs = x[start:start + bs]
        idx = xs.T.reshape(-1)
        rows = _sc_gather(table, idx)
        g3 = rows.reshape(SEQ, bs, EMBED)
        outs.append(_tc_mlp(g3, w1t, b1r, w2c, b2r))
        start += bs
    return jnp.concatenate(outs)


# speedup vs baseline: 1.0286x; 1.0286x over previous
import jax
import jax.numpy as jnp
from jax import lax
from jax.experimental import pallas as pl
from jax.experimental.pallas import tpu as pltpu
from jax.experimental.pallas import tpu_sc as plsc

VOCAB = 100000
EMBED = 128
SEQ = 10
HIDDEN = 1024
BATCH = 16384
NUM_IDX = BATCH * SEQ

NC, NS = 2, 16
NW = NC * NS

SLICES = (4096, 4096, 4096, 4096)
GROUND = 320


def _make_gather_body(b_per_w):
    nround = b_per_w // GROUND

    def _gather_body(table_hbm, idx_hbm, out_hbm, idx_v, rows_a, rows_b,
                     gsem_a, gsem_b, wsem_a, wsem_b):
        wid = lax.axis_index("s") * NC + lax.axis_index("c")
        base = wid * b_per_w

        bufs = [(rows_a, gsem_a, wsem_a), (rows_b, gsem_b, wsem_b)]
        gather = {}
        write = [None, None]

        def start_gather(k):
            rv, gs, _ = bufs[k % 2]
            gather[k] = pltpu.async_copy(
                table_hbm.at[idx_v.at[pl.ds(k * GROUND, GROUND)]], rv, gs)

        pltpu.sync_copy(idx_hbm.at[pl.ds(base, GROUND)],
                        idx_v.at[pl.ds(0, GROUND)])
        start_gather(0)
        if b_per_w > GROUND:
            pltpu.sync_copy(idx_hbm.at[pl.ds(base + GROUND, b_per_w - GROUND)],
                            idx_v.at[pl.ds(GROUND, b_per_w - GROUND)])
        for k in range(nround):
            rv, _, ws = bufs[k % 2]
            if k + 1 < nround:
                if write[(k + 1) % 2] is not None:
                    write[(k + 1) % 2].wait()
                start_gather(k + 1)
            gather[k].wait()
            write[k % 2] = pltpu.async_copy(
                rv, out_hbm.at[pl.ds(base + k * GROUND, GROUND)], ws)
        for w in write:
            if w is not None:
                w.wait()

    return _gather_body


def _sc_gather(table, idx):
    n_idx = idx.shape[0]
    b_per_w = n_idx // NW
    mesh = plsc.VectorSubcoreMesh(core_axis_name="c", subcore_axis_name="s")
    kfn = pl.kernel(
        _make_gather_body(b_per_w),
        mesh=mesh,
        out_type=jax.ShapeDtypeStruct((n_idx, EMBED), table.dtype),
        scratch_types=[
            pltpu.VMEM((b_per_w,), jnp.int32),
            pltpu.VMEM((GROUND, EMBED), table.dtype),
            pltpu.VMEM((GROUND, EMBED), table.dtype),
            pltpu.SemaphoreType.DMA,
            pltpu.SemaphoreType.DMA,
            pltpu.SemaphoreType.DMA,
            pltpu.SemaphoreType.DMA,
        ],
    )
    return kfn(table, idx)


def _mlp_body(g_ref, w1t_ref, b1_ref, w2c_ref, b2_ref, out_ref):
    a = jnp.concatenate([g_ref[s] for s in range(SEQ)], axis=-1)
    a = a.astype(jnp.bfloat16)
    h = jnp.dot(a, w1t_ref[...], preferred_element_type=jnp.float32)
    hb = jnp.maximum(h.astype(jnp.bfloat16) + b1_ref[...], 0)
    s128 = jnp.dot(hb, w2c_ref[...], preferred_element_type=jnp.float32)
    s = s128[:, 0] + b2_ref[0, 0]
    out_ref[...] = jax.nn.sigmoid(s)


def _tc_mlp(g3, w1t, b1, w2c, b2):
    batch_s = g3.shape[1]
    chunk_m = min(2048, batch_s)
    return pl.pallas_call(
        _mlp_body,
        grid=(batch_s // chunk_m,),
        in_specs=[
            pl.BlockSpec((SEQ, chunk_m, EMBED), lambda i: (0, i, 0)),
            pl.BlockSpec((SEQ * EMBED, HIDDEN), lambda i: (0, 0)),
            pl.BlockSpec((1, HIDDEN), lambda i: (0, 0)),
            pl.BlockSpec((HIDDEN, 128), lambda i: (0, 0)),
            pl.BlockSpec((1, 1), lambda i: (0, 0)),
        ],
        out_specs=pl.BlockSpec((chunk_m,), lambda i: (i,)),
        out_shape=jax.ShapeDtypeStruct((batch_s,), jnp.float32),
        compiler_params=pltpu.CompilerParams(
            dimension_semantics=("parallel",),
        ),
    )(g3, w1t, b1, w2c, b2)


def kernel(x, table, W1, b1, W2, b2):
    w1t = W1.T.astype(jnp.bfloat16)
    b1r = b1.astype(jnp.bfloat16).reshape(1, HIDDEN)
    w2c = jnp.zeros((HIDDEN, 128), jnp.float32).at[:, 0].set(W2[0])
    w2c = w2c.astype(jnp.bfloat16)
    b2r = b2.reshape(1, 1)
    outs = []
    start = 0
    for bs in SLICES:
        xs = x[start:start + bs]
        idx = xs.T.reshape(-1)
        rows = _sc_gather(table, idx)
        g3 = rows.reshape(SEQ, bs, EMBED)
        outs.append(_tc_mlp(g3, w1t, b1r, w2c, b2r))
        start += bs
    return jnp.concatenate(outs)
